# Initial kernel scaffold; baseline (speedup 1.0000x reference)
import jax
import jax.numpy as jnp
from jax.experimental import pallas as pl


def kernel(type_idx, card_idx, patron_idx, effect_idx, effect_amt, type_emb, patron_emb, effect_emb, card_table, W1, b1, W2, b2):
    raise NotImplementedError("write your pallas kernel here")


# speedup vs baseline: 4.1954x; 4.1954x over previous
import functools

import jax
import jax.numpy as jnp
from jax import lax
from jax.experimental import pallas as pl
from jax.experimental.pallas import tpu as pltpu
from jax.experimental.pallas import tpu_sc as plsc

_MAX_EFFECT_AMOUNT = 10.0
_B = 16384
_DM = 256
_FW = 128
_CARD_OFF = 35
_NW = 32
_BPW = _B // _NW


@functools.cache
def _make_card_gather():
    @functools.partial(
        pl.kernel,
        mesh=plsc.VectorSubcoreMesh(core_axis_name="c", subcore_axis_name="s"),
        out_type=jax.ShapeDtypeStruct((_B, _FW), jnp.float32),
        scratch_types=[
            pltpu.VMEM((_BPW,), jnp.int32),
            pltpu.VMEM((_BPW, _FW), jnp.float32),
            pltpu.SemaphoreType.DMA,
        ],
    )
    def _card_gather(table_hbm, idx_hbm, out_hbm, idx_v, rows_v, sem):
        wid = lax.axis_index("s") * 2 + lax.axis_index("c")
        base = wid * _BPW
        pltpu.sync_copy(idx_hbm.at[pl.ds(base, _BPW)], idx_v)
        pltpu.async_copy(table_hbm.at[idx_v], rows_v, sem).wait()
        pltpu.sync_copy(rows_v, out_hbm.at[pl.ds(base, _BPW)])

    return _card_gather


def _prep_body(type_emb, patron_emb, effect_emb, w1t, w1p, w1e, w_flag, b1,
               mt_out, mp_out, me_out):
    flag = (lax.broadcasted_iota(jnp.int32, (7, 1), 0) == 2).astype(jnp.float32)
    mt_out[...] = (jnp.dot(type_emb[...], w1t[...],
                           preferred_element_type=jnp.float32)
                   + b1[...] + flag * w_flag[...])
    mp_out[...] = jnp.dot(patron_emb[...], w1p[...],
                          preferred_element_type=jnp.float32)
    me_out[...] = jnp.dot(effect_emb[...], w1e[...],
                          preferred_element_type=jnp.float32)


def _main_body(t_ref, p_ref, e_ref, amt_ref, card_ref, m_ref, w2_ref, b2_ref,
               out_ref):
    t = t_ref[...]
    p = p_ref[...]
    e = e_ref[...]
    amt = amt_ref[...]
    blk = t.shape[0]
    col = lax.broadcasted_iota(jnp.int32, (blk, _FW), 1)
    f_t = (col == t).astype(jnp.float32)
    f_p = ((col - 7 == p) & (t == 4)).astype(jnp.float32)
    scale = 1.0 + amt / _MAX_EFFECT_AMOUNT
    f_e = ((col - 17 == e) & (t == 5)).astype(jnp.float32) * scale
    card_mask = (t <= 3).astype(jnp.float32)
    feat = f_t + f_p + f_e + card_ref[...] * card_mask
    h = jnp.maximum(jnp.dot(feat, m_ref[...],
                            preferred_element_type=jnp.float32), 0.0)
    out_ref[...] = jnp.dot(h, w2_ref[...],
                           preferred_element_type=jnp.float32) + b2_ref[...]


def kernel(type_idx, card_idx, patron_idx, effect_idx, effect_amt,
           type_emb, patron_emb, effect_emb, card_table, W1, b1, W2, b2):
    f32 = jnp.float32

    w1t = W1[0:256]
    w1c = W1[256:321]
    w1p = W1[321:331]
    w1e = W1[331:587]
    w_flag = W1[587:588]
    card_pad = jnp.zeros((card_table.shape[0], _FW), f32)
    card_pad = card_pad.at[:, _CARD_OFF:_CARD_OFF + 65].set(card_table)

    mt, mp, me = pl.pallas_call(
        _prep_body,
        out_shape=[
            jax.ShapeDtypeStruct((7, _DM), f32),
            jax.ShapeDtypeStruct((10, _DM), f32),
            jax.ShapeDtypeStruct((18, _DM), f32),
        ],
    )(type_emb, patron_emb, effect_emb, w1t, w1p, w1e, w_flag,
      b1.reshape(1, _DM))

    m = jnp.concatenate(
        [mt, mp, me, w1c, jnp.zeros((_FW - 100, _DM), f32)], axis=0)

    card_rows = _make_card_gather()(card_pad, card_idx)

    blk = 1024
    nblk = _B // blk
    row_spec = pl.BlockSpec((blk, 1), lambda i: (i, 0))
    out = pl.pallas_call(
        _main_body,
        grid=(nblk,),
        in_specs=[
            row_spec, row_spec, row_spec, row_spec,
            pl.BlockSpec((blk, _FW), lambda i: (i, 0)),
            pl.BlockSpec((_FW, _DM), lambda i: (0, 0)),
            pl.BlockSpec((_DM, _DM), lambda i: (0, 0)),
            pl.BlockSpec((1, _DM), lambda i: (0, 0)),
        ],
        out_specs=pl.BlockSpec((blk, _DM), lambda i: (i, 0)),
        out_shape=jax.ShapeDtypeStruct((_B, _DM), f32),
    )(type_idx.reshape(_B, 1), patron_idx.reshape(_B, 1),
      effect_idx.reshape(_B, 1), effect_amt.reshape(_B, 1),
      card_rows, m, W2, b2.reshape(1, _DM))
    return out
